# Initial kernel scaffold; baseline (speedup 1.0000x reference)
import jax
import jax.numpy as jnp
from jax.experimental import pallas as pl


def kernel(enc, dec, codebook):
    raise NotImplementedError("write your pallas kernel here")


# speedup vs baseline: 4.2298x; 4.2298x over previous
import functools

import jax
import jax.numpy as jnp
from jax import lax
from jax.experimental import pallas as pl
from jax.experimental.pallas import tpu as pltpu
from jax.experimental.pallas import tpu_sc as plsc

_B, _N, _C = 8, 1024, 384
_G, _K, _D = 6, 1024, 64
_ROWS = _B * _N
_TR = 512
_GRID = _ROWS // _TR
_RG = _ROWS * _G
_DENOM = float(_ROWS * _G * _D)

_NC, _NS = 2, 16
_NW = _NC * _NS
_CHUNK = _RG // _NW
_BATCH = 128
_NB = _CHUNK // _BATCH
_LANES = 16


def _dist_body(enc_ref, dec_ref, cbt_ref, codes_ref, idxo_ref, losssum_ref):
    step = pl.program_id(0)
    z = enc_ref[...] - dec_ref[...]
    total = jnp.zeros((), jnp.float32)
    for g in range(_G):
        zg = z[:, g * _D:(g + 1) * _D]
        cbt = cbt_ref[g]
        cross = lax.dot_general(
            zg, cbt, (((1,), (0,)), ((), ())),
            preferred_element_type=jnp.float32,
            precision=lax.Precision.DEFAULT)
        zsq = jnp.sum(zg * zg, axis=1, keepdims=True)
        cbsq = jnp.sum(cbt * cbt, axis=0, keepdims=True)
        dist = (zsq - 2.0 * cross) + cbsq
        m = jnp.min(dist, axis=1, keepdims=True)
        ii = lax.broadcasted_iota(jnp.int32, (_TR, _K), 1)
        code = jnp.min(jnp.where(dist == m, ii, jnp.int32(_K)), axis=1)
        codes_ref[g, :] = code
        idxo_ref[g, :] = code + jnp.int32(g * _K)
        total += jnp.sum(m)

    total2d = total[None, None]

    @pl.when(step == 0)
    def _():
        losssum_ref[...] = total2d

    @pl.when(step != 0)
    def _():
        losssum_ref[...] += total2d


def _distances(enc2d, dec2d, cbt):
    return pl.pallas_call(
        _dist_body,
        grid=(_GRID,),
        in_specs=[
            pl.BlockSpec((_TR, _C), lambda i: (i, 0)),
            pl.BlockSpec((_TR, _C), lambda i: (i, 0)),
            pl.BlockSpec((_G, _D, _K), lambda i: (0, 0, 0)),
        ],
        out_specs=[
            pl.BlockSpec((_G, _TR), lambda i: (0, i)),
            pl.BlockSpec((_G, _TR), lambda i: (0, i)),
            pl.BlockSpec((1, 1), lambda i: (0, 0)),
        ],
        out_shape=[
            jax.ShapeDtypeStruct((_G, _ROWS), jnp.int32),
            jax.ShapeDtypeStruct((_G, _ROWS), jnp.int32),
            jax.ShapeDtypeStruct((1, 1), jnp.float32),
        ],
    )(enc2d, dec2d, cbt)


def _gather_fuse_body(idx_hbm, dec_hbm, cb_hbm, out_hbm, idx_v, rows_v, dec_v, sem):
    wid = lax.axis_index("s") * _NC + lax.axis_index("c")
    base = wid * _CHUNK
    pltpu.sync_copy(idx_hbm.at[wid], idx_v)
    for b in range(_NB):
        off = base + b * _BATCH
        gather = pltpu.async_copy(cb_hbm.at[idx_v.at[b]], rows_v, sem)
        pltpu.sync_copy(dec_hbm.at[pl.ds(off, _BATCH)], dec_v)
        gather.wait()

        def _add_row(i, carry):
            for q in range(_D // _LANES):
                sl = pl.ds(q * _LANES, _LANES)
                rows_v[i, sl] = rows_v[i, sl] + dec_v[i, sl]
            return carry

        lax.fori_loop(0, _BATCH, _add_row, 0)
        pltpu.sync_copy(rows_v, out_hbm.at[pl.ds(off, _BATCH)])


@functools.cache
def _gather_fuse():
    return pl.kernel(
        _gather_fuse_body,
        mesh=plsc.VectorSubcoreMesh(core_axis_name="c", subcore_axis_name="s"),
        compiler_params=pltpu.CompilerParams(use_tc_tiling_on_sc=False),
        out_type=jax.ShapeDtypeStruct((_RG, _D), jnp.float32),
        scratch_types=[
            pltpu.VMEM((_NB, _BATCH), jnp.int32),
            pltpu.VMEM((_BATCH, _D), jnp.float32),
            pltpu.VMEM((_BATCH, _D), jnp.float32),
            pltpu.SemaphoreType.DMA,
        ],
    )


def kernel(enc, dec, codebook):
    enc2d = enc.reshape(_ROWS, _C)
    dec2d = dec.reshape(_ROWS, _C)
    cbt = jnp.transpose(codebook, (0, 2, 1))
    codes_gt, idxo_gt, losssum = _distances(enc2d, dec2d, cbt)

    codes = codes_gt.T.reshape(_B, _N, _G)
    idx = idxo_gt.T.reshape(_NW, _NB, _BATCH)
    cb_flat = codebook.reshape(_G * _K, _D)
    dec_rows = dec2d.reshape(_RG, _D)

    fused = _gather_fuse()(idx, dec_rows, cb_flat)
    dec_refine = fused.reshape(_B, _N, _C)

    loss = losssum[0, 0] / _DENOM
    return (dec_refine, loss, loss, codes)


# speedup vs baseline: 5.4188x; 1.2811x over previous
import functools

import jax
import jax.numpy as jnp
from jax import lax
from jax.experimental import pallas as pl
from jax.experimental.pallas import tpu as pltpu
from jax.experimental.pallas import tpu_sc as plsc

_B, _N, _C = 8, 1024, 384
_G, _K, _D = 6, 1024, 64
_ROWS = _B * _N
_TR = 512
_GRID = _ROWS // _TR
_RG = _ROWS * _G
_DENOM = float(_ROWS * _G * _D)

_NC, _NS = 2, 16
_NW = _NC * _NS
_CHUNK = _RG // _NW
_BATCH = 128
_NB = _CHUNK // _BATCH
_LANES = 16


_KC = 128
_NKC = _K // _KC


def _dist_body(enc_ref, dec_ref, cbt_ref, codes_ref, losssum_ref):
    step = pl.program_id(0)
    z = enc_ref[...] - dec_ref[...]
    lanef = lax.broadcasted_iota(jnp.int32, (_TR, _KC), 1).astype(jnp.float32)
    total = jnp.zeros((), jnp.float32)
    code_cols = []
    for g in range(_G):
        zg = z[:, g * _D:(g + 1) * _D]
        cbt = cbt_ref[g]
        cross = lax.dot_general(
            zg, cbt, (((1,), (0,)), ((), ())),
            preferred_element_type=jnp.float32,
            precision=lax.Precision.DEFAULT)
        zsq = jnp.sum(zg * zg, axis=1, keepdims=True)
        cbsq = jnp.sum(cbt * cbt, axis=0, keepdims=True)
        mval = None
        midxf = None
        for j in range(_NKC):
            cj = cross[:, j * _KC:(j + 1) * _KC]
            dj = (zsq - 2.0 * cj) + cbsq[:, j * _KC:(j + 1) * _KC]
            if j == 0:
                mval = dj
                midxf = jnp.zeros((_TR, _KC), jnp.float32)
            else:
                better = dj < mval
                mval = jnp.where(better, dj, mval)
                midxf = jnp.where(better, jnp.float32(j), midxf)
        m = jnp.min(mval, axis=1, keepdims=True)
        kf = jnp.where(mval == m, midxf * jnp.float32(_KC) + lanef,
                       jnp.float32(_K))
        code_cols.append(jnp.min(kf, axis=1, keepdims=True))
        total += jnp.sum(m)

    codes_ref[...] = jnp.concatenate(code_cols, axis=1).astype(jnp.int32)

    total2d = total[None, None]

    @pl.when(step == 0)
    def _():
        losssum_ref[...] = total2d

    @pl.when(step != 0)
    def _():
        losssum_ref[...] += total2d


def _distances(enc2d, dec2d, cbt):
    return pl.pallas_call(
        _dist_body,
        grid=(_GRID,),
        in_specs=[
            pl.BlockSpec((_TR, _C), lambda i: (i, 0)),
            pl.BlockSpec((_TR, _C), lambda i: (i, 0)),
            pl.BlockSpec((_G, _D, _K), lambda i: (0, 0, 0)),
        ],
        out_specs=[
            pl.BlockSpec((_TR, _G), lambda i: (i, 0)),
            pl.BlockSpec((1, 1), lambda i: (0, 0)),
        ],
        out_shape=[
            jax.ShapeDtypeStruct((_ROWS, _G), jnp.int32),
            jax.ShapeDtypeStruct((1, 1), jnp.float32),
        ],
    )(enc2d, dec2d, cbt)


def _gather_fuse_body(idx_hbm, dec_hbm, cb_hbm, out_hbm, idx_v, rows_v, dec_v, sem):
    wid = lax.axis_index("s") * _NC + lax.axis_index("c")
    base = wid * _CHUNK
    pltpu.sync_copy(idx_hbm.at[wid], idx_v)
    lane = lax.iota(jnp.int32, _LANES)
    for b in range(_NB):
        off = base + b * _BATCH
        for q in range(_BATCH // _LANES):
            sl = pl.ds(q * _LANES, _LANES)
            rv = lane + jnp.int32(off + q * _LANES)
            idx_v[b, sl] = idx_v[b, sl] + lax.rem(rv, jnp.int32(_G)) * jnp.int32(_K)
        gather = pltpu.async_copy(cb_hbm.at[idx_v.at[b]], rows_v, sem)
        pltpu.sync_copy(dec_hbm.at[pl.ds(off, _BATCH)], dec_v)
        gather.wait()

        def _add_row(i, carry):
            for q in range(_D // _LANES):
                sl = pl.ds(q * _LANES, _LANES)
                rows_v[i, sl] = rows_v[i, sl] + dec_v[i, sl]
            return carry

        lax.fori_loop(0, _BATCH, _add_row, 0)
        pltpu.sync_copy(rows_v, out_hbm.at[pl.ds(off, _BATCH)])


@functools.cache
def _gather_fuse():
    return pl.kernel(
        _gather_fuse_body,
        mesh=plsc.VectorSubcoreMesh(core_axis_name="c", subcore_axis_name="s"),
        compiler_params=pltpu.CompilerParams(use_tc_tiling_on_sc=False),
        out_type=jax.ShapeDtypeStruct((_RG, _D), jnp.float32),
        scratch_types=[
            pltpu.VMEM((_NB, _BATCH), jnp.int32),
            pltpu.VMEM((_BATCH, _D), jnp.float32),
            pltpu.VMEM((_BATCH, _D), jnp.float32),
            pltpu.SemaphoreType.DMA,
        ],
    )


def kernel(enc, dec, codebook):
    enc2d = enc.reshape(_ROWS, _C)
    dec2d = dec.reshape(_ROWS, _C)
    cbt = jnp.transpose(codebook, (0, 2, 1))
    codes_rg, losssum = _distances(enc2d, dec2d, cbt)

    codes = codes_rg.reshape(_B, _N, _G)
    idx = codes_rg.reshape(_NW, _NB, _BATCH)
    cb_flat = codebook.reshape(_G * _K, _D)
    dec_rows = dec2d.reshape(_RG, _D)

    fused = _gather_fuse()(idx, dec_rows, cb_flat)
    dec_refine = fused.reshape(_B, _N, _C)

    loss = losssum[0, 0] / _DENOM
    return (dec_refine, loss, loss, codes)


# speedup vs baseline: 6.0079x; 1.1087x over previous
import functools

import jax
import jax.numpy as jnp
from jax import lax
from jax.experimental import pallas as pl
from jax.experimental.pallas import tpu as pltpu
from jax.experimental.pallas import tpu_sc as plsc

_B, _N, _C = 8, 1024, 384
_G, _K, _D = 6, 1024, 64
_ROWS = _B * _N
_TR = 512
_GRID = _ROWS // _TR
_RG = _ROWS * _G
_DENOM = float(_ROWS * _G * _D)

_NC, _NS = 2, 16
_NW = _NC * _NS
_CHUNK = _RG // _NW
_BATCH = 128
_NB = _CHUNK // _BATCH
_LANES = 16


_KC = 128
_NKC = _K // _KC


def _dist_body(enc_ref, dec_ref, cbt_ref, codes_ref, losssum_ref):
    step = pl.program_id(0)
    z = enc_ref[...] - dec_ref[...]
    lanef = lax.broadcasted_iota(jnp.int32, (_TR, _KC), 1).astype(jnp.float32)
    total = jnp.zeros((), jnp.float32)
    code_cols = []
    for g in range(_G):
        zg = z[:, g * _D:(g + 1) * _D]
        cbt = cbt_ref[g]
        cross = lax.dot_general(
            zg, cbt, (((1,), (0,)), ((), ())),
            preferred_element_type=jnp.float32,
            precision=lax.Precision.DEFAULT)
        zsq = jnp.sum(zg * zg, axis=1, keepdims=True)
        cbsq = jnp.sum(cbt * cbt, axis=0, keepdims=True)
        mval = None
        midxf = None
        for j in range(_NKC):
            cj = cross[:, j * _KC:(j + 1) * _KC]
            dj = (zsq - 2.0 * cj) + cbsq[:, j * _KC:(j + 1) * _KC]
            if j == 0:
                mval = dj
                midxf = jnp.zeros((_TR, _KC), jnp.float32)
            else:
                better = dj < mval
                mval = jnp.where(better, dj, mval)
                midxf = jnp.where(better, jnp.float32(j), midxf)
        m = jnp.min(mval, axis=1, keepdims=True)
        kf = jnp.where(mval == m, midxf * jnp.float32(_KC) + lanef,
                       jnp.float32(_K))
        code_cols.append(jnp.min(kf, axis=1, keepdims=True))
        total += jnp.sum(m)

    codes_ref[...] = jnp.concatenate(code_cols, axis=1).astype(jnp.int32)

    total2d = total[None, None]

    @pl.when(step == 0)
    def _():
        losssum_ref[...] = total2d

    @pl.when(step != 0)
    def _():
        losssum_ref[...] += total2d


def _distances(enc2d, dec2d, cbt):
    return pl.pallas_call(
        _dist_body,
        grid=(_GRID,),
        in_specs=[
            pl.BlockSpec((_TR, _C), lambda i: (i, 0)),
            pl.BlockSpec((_TR, _C), lambda i: (i, 0)),
            pl.BlockSpec((_G, _D, _K), lambda i: (0, 0, 0)),
        ],
        out_specs=[
            pl.BlockSpec((_TR, _G), lambda i: (i, 0)),
            pl.BlockSpec((1, 1), lambda i: (0, 0)),
        ],
        out_shape=[
            jax.ShapeDtypeStruct((_ROWS, _G), jnp.int32),
            jax.ShapeDtypeStruct((1, 1), jnp.float32),
        ],
    )(enc2d, dec2d, cbt)


def _gather_body(idx_hbm, cb_hbm, out_hbm, idx_v, rows_v, sem):
    wid = lax.axis_index("s") * _NC + lax.axis_index("c")
    base = wid * _CHUNK
    pltpu.sync_copy(idx_hbm.at[wid], idx_v)
    lane = lax.iota(jnp.int32, _LANES)
    for b in range(_NB):
        for q in range(_BATCH // _LANES):
            sl = pl.ds(q * _LANES, _LANES)
            rv = lane + jnp.int32(base + b * _BATCH + q * _LANES)
            idx_v[b, sl] = idx_v[b, sl] + lax.rem(rv, jnp.int32(_G)) * jnp.int32(_K)
    copies = [
        pltpu.async_copy(cb_hbm.at[idx_v.at[b]],
                         rows_v.at[pl.ds(b * _BATCH, _BATCH)], sem)
        for b in range(_NB)
    ]
    for c in copies:
        c.wait()
    pltpu.sync_copy(rows_v, out_hbm.at[pl.ds(base, _CHUNK)])


@functools.cache
def _gather():
    return pl.kernel(
        _gather_body,
        mesh=plsc.VectorSubcoreMesh(core_axis_name="c", subcore_axis_name="s"),
        compiler_params=pltpu.CompilerParams(use_tc_tiling_on_sc=False),
        out_type=jax.ShapeDtypeStruct((_RG, _D), jnp.float32),
        scratch_types=[
            pltpu.VMEM((_NB, _BATCH), jnp.int32),
            pltpu.VMEM((_CHUNK, _D), jnp.float32),
            pltpu.SemaphoreType.DMA,
        ],
    )


def kernel(enc, dec, codebook):
    enc2d = enc.reshape(_ROWS, _C)
    dec2d = dec.reshape(_ROWS, _C)
    cbt = jnp.transpose(codebook, (0, 2, 1))
    codes_rg, losssum = _distances(enc2d, dec2d, cbt)

    codes = codes_rg.reshape(_B, _N, _G)
    idx = codes_rg.reshape(_NW, _NB, _BATCH)
    cb_flat = codebook.reshape(_G * _K, _D)

    zq = _gather()(idx, cb_flat)
    dec_refine = (dec2d + zq.reshape(_ROWS, _C)).reshape(_B, _N, _C)

    loss = losssum[0, 0] / _DENOM
    return (dec_refine, loss, loss, codes)


# speedup vs baseline: 7.1974x; 1.1980x over previous
import functools

import jax
import jax.numpy as jnp
from jax import lax
from jax.experimental import pallas as pl
from jax.experimental.pallas import tpu as pltpu
from jax.experimental.pallas import tpu_sc as plsc

_B, _N, _C = 8, 1024, 384
_G, _K, _D = 6, 1024, 64
_ROWS = _B * _N
_TR = 512
_GRID = _ROWS // _TR
_RG = _ROWS * _G
_DENOM = float(_ROWS * _G * _D)

_NC, _NS = 2, 16
_NW = _NC * _NS
_CHUNK = _RG // _NW
_BATCH = 128
_NB = _CHUNK // _BATCH
_LANES = 16


_KC = 128
_NKC = _K // _KC


def _dist_body(enc_ref, dec_ref, cbt2_ref, codes_ref, losssum_ref):
    step = pl.program_id(0)
    z = enc_ref[...] - dec_ref[...]
    lanef = lax.broadcasted_iota(jnp.int32, (_TR, _KC), 1).astype(jnp.float32)
    total = jnp.sum(z * z)
    code_cols = []
    for g in range(_G):
        zg = z[:, g * _D:(g + 1) * _D]
        cbt2 = cbt2_ref[g]
        cross2 = lax.dot_general(
            zg, cbt2, (((1,), (0,)), ((), ())),
            preferred_element_type=jnp.float32,
            precision=lax.Precision.DEFAULT)
        cbsq = 0.25 * jnp.sum(cbt2 * cbt2, axis=0, keepdims=True)
        mval = None
        midxf = None
        for j in range(_NKC):
            sl = slice(j * _KC, (j + 1) * _KC)
            dj = cross2[:, sl] + cbsq[:, sl]
            if j == 0:
                mval = dj
                midxf = jnp.zeros((_TR, _KC), jnp.float32)
            else:
                better = dj < mval
                mval = jnp.where(better, dj, mval)
                midxf = jnp.where(better, jnp.float32(j), midxf)
        m = jnp.min(mval, axis=1, keepdims=True)
        kf = jnp.where(mval == m, midxf * jnp.float32(_KC) + lanef,
                       jnp.float32(_K))
        code_cols.append(jnp.min(kf, axis=1, keepdims=True))
        total += jnp.sum(m)

    codes_ref[...] = jnp.concatenate(code_cols, axis=1).astype(jnp.int32)

    total2d = total[None, None]

    @pl.when(step == 0)
    def _():
        losssum_ref[...] = total2d

    @pl.when(step != 0)
    def _():
        losssum_ref[...] += total2d


def _distances(enc2d, dec2d, cbt):
    return pl.pallas_call(
        _dist_body,
        grid=(_GRID,),
        in_specs=[
            pl.BlockSpec((_TR, _C), lambda i: (i, 0)),
            pl.BlockSpec((_TR, _C), lambda i: (i, 0)),
            pl.BlockSpec((_G, _D, _K), lambda i: (0, 0, 0)),
        ],
        out_specs=[
            pl.BlockSpec((_TR, _G), lambda i: (i, 0)),
            pl.BlockSpec((1, 1), lambda i: (0, 0)),
        ],
        out_shape=[
            jax.ShapeDtypeStruct((_ROWS, _G), jnp.int32),
            jax.ShapeDtypeStruct((1, 1), jnp.float32),
        ],
    )(enc2d, dec2d, cbt)


def _gather_body(idx_hbm, cb_hbm, out_hbm, idx_v, rows_v, sem, out_sem):
    wid = lax.axis_index("s") * _NC + lax.axis_index("c")
    base = wid * _CHUNK
    pltpu.sync_copy(idx_hbm.at[wid], idx_v)
    lane = lax.iota(jnp.int32, _LANES)
    for b in range(_NB):
        for q in range(_BATCH // _LANES):
            sl = pl.ds(q * _LANES, _LANES)
            rv = lane + jnp.int32(base + b * _BATCH + q * _LANES)
            idx_v[b, sl] = idx_v[b, sl] + lax.rem(rv, jnp.int32(_G)) * jnp.int32(_K)
    copies = [
        pltpu.async_copy(cb_hbm.at[idx_v.at[b]],
                         rows_v.at[pl.ds(b * _BATCH, _BATCH)], sem)
        for b in range(_NB)
    ]
    outs = []
    for b, c in enumerate(copies):
        c.wait()
        outs.append(pltpu.async_copy(
            rows_v.at[pl.ds(b * _BATCH, _BATCH)],
            out_hbm.at[pl.ds(base + b * _BATCH, _BATCH)], out_sem))
    for o in outs:
        o.wait()


@functools.cache
def _gather():
    return pl.kernel(
        _gather_body,
        mesh=plsc.VectorSubcoreMesh(core_axis_name="c", subcore_axis_name="s"),
        compiler_params=pltpu.CompilerParams(use_tc_tiling_on_sc=False),
        out_type=jax.ShapeDtypeStruct((_RG, _D), jnp.float32),
        scratch_types=[
            pltpu.VMEM((_NB, _BATCH), jnp.int32),
            pltpu.VMEM((_CHUNK, _D), jnp.float32),
            pltpu.SemaphoreType.DMA,
            pltpu.SemaphoreType.DMA,
        ],
    )


def kernel(enc, dec, codebook):
    enc2d = enc.reshape(_ROWS, _C)
    dec2d = dec.reshape(_ROWS, _C)
    cbt = -2.0 * jnp.transpose(codebook, (0, 2, 1))
    codes_rg, losssum = _distances(enc2d, dec2d, cbt)

    codes = codes_rg.reshape(_B, _N, _G)
    idx = codes_rg.reshape(_NW, _NB, _BATCH)
    cb_flat = codebook.reshape(_G * _K, _D)

    zq = _gather()(idx, cb_flat)
    dec_refine = (dec2d + zq.reshape(_ROWS, _C)).reshape(_B, _N, _C)

    loss = losssum[0, 0] / _DENOM
    return (dec_refine, loss, loss, codes)
